# baseline (speedup 1.0000x reference)
import functools

import jax
import jax.numpy as jnp
from jax import lax
from jax.experimental import pallas as pl
from jax.experimental.pallas import tpu as pltpu
from jax.experimental.pallas import tpu_sc as plsc

N = 10000
D = 128
E = 320000
L = 100000
NC, NS, LANES = 2, 16, 16
NW = NC * NS

ROWS_PER_TILE = 632
NPAD = NS * ROWS_PER_TILE
HD = 64
CHUNK = 128
NCHUNK_E = 160
EP = NS * NCHUNK_E * CHUNK
NCHUNK_L = 28
LP = NW * NCHUNK_L * CHUNK


def _make_segsum(with_counts):
    mesh = plsc.VectorSubcoreMesh(core_axis_name="c", subcore_axis_name="s")
    out_type = [jax.ShapeDtypeStruct((NC, NPAD, HD), jnp.float32),
                jax.ShapeDtypeStruct((NC, NPAD, HD), jnp.float32)]
    NSLOT = 8
    DEPTH = 4
    G = 16
    NBLK = NCHUNK_E // G
    scratch = [pltpu.VMEM((G, CHUNK), jnp.int32) for _ in range(4)]
    scratch += [pltpu.VMEM((CHUNK, HD), jnp.float32) for _ in range(NSLOT)]
    scratch += [
        pltpu.VMEM_SHARED((NPAD, HD), jnp.float32),
        pltpu.SemaphoreType.DMA((NSLOT,)),
        pltpu.SemaphoreType.DMA((NSLOT,)),
        pltpu.SemaphoreType.DMA((2,)),
    ]
    if with_counts:
        out_type.append(jax.ShapeDtypeStruct((NC, NPAD, LANES), jnp.float32))
        scratch += [
            pltpu.VMEM((CHUNK, LANES), jnp.float32),
            pltpu.VMEM_SHARED((NPAD, LANES), jnp.float32),
            pltpu.SemaphoreType.DMA((NSLOT,)),
        ]

    def body(*args):
        if with_counts:
            (tabL, tabR, srci, dsti, zf, zc, ones_h, sums_l, sums_r, cnt,
             sv0, sv1, dv0, dv1,
             b0, b1, b2, b3, b4, b5, b6, b7, acc_sh, gsem, ssem, isem,
             ones_v, cnt_sh, csem) = args
        else:
            (tabL, tabR, srci, dsti, zf, sums_l, sums_r, sv0, sv1, dv0, dv1,
             b0, b1, b2, b3, b4, b5, b6, b7, acc_sh, gsem, ssem, isem) = args
        srcv = (sv0, sv1)
        dstv = (dv0, dv1)
        bufs = (b0, b1, b2, b3, b4, b5, b6, b7)
        cid = lax.axis_index("c")
        sid = lax.axis_index("s")
        base = pl.multiple_of(sid * ROWS_PER_TILE, 8)
        sl = pl.ds(base, ROWS_PER_TILE)

        def i_fire(blk, islot):
            off = pl.multiple_of(blk * G, 8)
            pltpu.async_copy(srci.at[cid, sid, pl.ds(off, G)], srcv[islot],
                             isem.at[islot])
            pltpu.async_copy(dsti.at[cid, sid, pl.ds(off, G)], dstv[islot],
                             isem.at[islot])

        def i_wait(islot):
            pltpu.make_async_copy(srci.at[0, 0, pl.ds(0, G)], srcv[islot],
                                  isem.at[islot]).wait()
            pltpu.make_async_copy(srci.at[0, 0, pl.ds(0, G)], dstv[islot],
                                  isem.at[islot]).wait()

        def run_half(tab, sums, do_cnt):
            def g_wait(b):
                pltpu.make_async_copy(tab.at[pl.ds(0, CHUNK)], bufs[b],
                                      gsem.at[b]).wait()

            def s_wait(b):
                pltpu.make_async_copy(tab.at[pl.ds(0, CHUNK)], bufs[b],
                                      ssem.at[b]).wait()

            def c_wait(b):
                pltpu.make_async_copy(zc.at[pl.ds(0, CHUNK)], ones_v,
                                      csem.at[b]).wait()

            def g_fire(islot, row, b):
                pltpu.async_copy(tab.at[srcv[islot].at[row]], bufs[b],
                                 gsem.at[b])

            def s_fire(islot, row, b):
                pltpu.async_copy(bufs[b], acc_sh.at[dstv[islot].at[row]],
                                 ssem.at[b], add=True)
                if do_cnt:
                    pltpu.async_copy(ones_v, cnt_sh.at[dstv[islot].at[row]],
                                     csem.at[b], add=True)

            pltpu.sync_copy(zf.at[sl], acc_sh.at[sl])
            if do_cnt:
                pltpu.sync_copy(zc.at[sl], cnt_sh.at[sl])
                pltpu.sync_copy(ones_h, ones_v)
            i_fire(0, 0)
            plsc.subcore_barrier()
            i_wait(0)
            for p in range(DEPTH):
                g_fire(0, p, p)

            def pair(bp, carry):
                for pb in range(2):
                    blk = bp * 2 + pb
                    for p in range(G):
                        b = p % NSLOT
                        nb = (b + DEPTH) % NSLOT
                        ci = blk * G + p
                        g_wait(b)
                        s_fire(pb, p, b)

                        @pl.when(ci >= DEPTH)
                        def _():
                            s_wait(nb)
                            if do_cnt:
                                c_wait(nb)

                        if p == 4:
                            @pl.when(blk <= NBLK - 2)
                            def _():
                                i_fire(blk + 1, 1 - pb)
                        if p == 11:
                            @pl.when(blk <= NBLK - 2)
                            def _():
                                i_wait(1 - pb)
                        tp = p + DEPTH
                        gslot, grow = (pb, tp) if tp < G else (1 - pb, tp - G)

                        @pl.when(ci <= NCHUNK_E - 1 - DEPTH)
                        def _():
                            g_fire(gslot, grow, nb)
                return carry

            lax.fori_loop(0, NBLK // 2, pair, 0)
            for b in range(NSLOT - DEPTH, NSLOT):
                s_wait(b)
                if do_cnt:
                    c_wait(b)
            plsc.subcore_barrier()
            pltpu.sync_copy(acc_sh.at[sl], sums.at[cid, sl])
            if do_cnt:
                pltpu.sync_copy(cnt_sh.at[sl], cnt.at[cid, sl])

        run_half(tabL, sums_l, with_counts)
        run_half(tabR, sums_r, False)

    return pl.kernel(body, out_type=tuple(out_type), mesh=mesh,
                     scratch_types=scratch,
                     compiler_params=pltpu.CompilerParams(
                         use_tc_tiling_on_sc=False))


_segsum_wc = _make_segsum(True)
_segsum_nc = _make_segsum(False)


def _make_labels():
    mesh = plsc.VectorSubcoreMesh(core_axis_name="c", subcore_axis_name="s")
    out_type = (jax.ShapeDtypeStruct((LP, D), jnp.float32),
                jax.ShapeDtypeStruct((LP, D), jnp.float32))
    NSLOT = 2
    DEPTH = 1
    scratch = [
        pltpu.VMEM((NCHUNK_L, CHUNK), jnp.int32),
        pltpu.VMEM((NCHUNK_L, CHUNK), jnp.int32),
    ]
    scratch += [pltpu.VMEM((CHUNK, D), jnp.float32) for _ in range(2 * NSLOT)]
    scratch += [
        pltpu.SemaphoreType.DMA((NSLOT,)),
        pltpu.SemaphoreType.DMA((NSLOT,)),
        pltpu.SemaphoreType.DMA((NSLOT,)),
        pltpu.SemaphoreType.DMA((NSLOT,)),
    ]

    def body(tab, l0i, l1i, uf, jf, l0_v, l1_v, ru0, ru1,
             rj0, rj1, gusem, gjsem, wusem, wjsem):
        cid = lax.axis_index("c")
        sid = lax.axis_index("s")
        w = sid * NC + cid
        rus = (ru0, ru1)
        rjs = (rj0, rj1)
        pltpu.sync_copy(l0i.at[w], l0_v)
        pltpu.sync_copy(l1i.at[w], l1_v)

        def g_fire(ci, b):
            pltpu.async_copy(tab.at[l0_v.at[ci]], rus[b], gusem.at[b])
            pltpu.async_copy(tab.at[l1_v.at[ci]], rjs[b], gjsem.at[b])

        def g_wait(b):
            pltpu.make_async_copy(tab.at[pl.ds(0, CHUNK)], rus[b],
                                  gusem.at[b]).wait()
            pltpu.make_async_copy(tab.at[pl.ds(0, CHUNK)], rjs[b],
                                  gjsem.at[b]).wait()

        def w_fire(ci, b):
            rb = pl.multiple_of(w * (NCHUNK_L * CHUNK) + ci * CHUNK, 8)
            pltpu.async_copy(rus[b], uf.at[pl.ds(rb, CHUNK)], wusem.at[b])
            pltpu.async_copy(rjs[b], jf.at[pl.ds(rb, CHUNK)], wjsem.at[b])

        def w_wait(b):
            pltpu.make_async_copy(tab.at[pl.ds(0, CHUNK)], rus[b],
                                  wusem.at[b]).wait()
            pltpu.make_async_copy(tab.at[pl.ds(0, CHUNK)], rjs[b],
                                  wjsem.at[b]).wait()

        for p in range(DEPTH):
            g_fire(p, p)

        def outer(io, carry):
            i = io * NSLOT
            for b in range(NSLOT):
                ci = i + b
                nb = (b + DEPTH) % NSLOT
                g_wait(b)
                w_fire(ci, b)

                @pl.when(ci >= DEPTH)
                def _():
                    w_wait(nb)

                @pl.when(ci <= NCHUNK_L - 1 - DEPTH)
                def _():
                    g_fire(ci + DEPTH, nb)
            return carry

        lax.fori_loop(0, NCHUNK_L // NSLOT, outer, 0)
        for b in range(NSLOT - DEPTH, NSLOT):
            w_wait(b)

    return pl.kernel(body, out_type=out_type, mesh=mesh,
                     scratch_types=scratch)


_labels = _make_labels()


def _mask_pad(y):
    rid = lax.broadcasted_iota(jnp.int32, y.shape, 0)
    return jnp.where(rid < N, y, 0.0)


def _enc_one(x, w, b, g, bb):
    h = jnp.dot(x, w, preferred_element_type=jnp.float32) + b
    hs = h[:N]
    mu = jnp.mean(hs, axis=0, keepdims=True)
    var = jnp.mean((hs - mu) ** 2, axis=0, keepdims=True)
    y = (h - mu) * lax.rsqrt(var + 1e-5) * g + bb
    return _mask_pad(jnp.maximum(y, 0.0))


def _encoder_body(x_ref, w_ref, b_ref, g_ref, bb_ref, o_ref):
    o_ref[...] = _enc_one(x_ref[...], w_ref[...], b_ref[...], g_ref[...],
                          bb_ref[...])


def _encoder(x, w, b, g, bb):
    return pl.pallas_call(
        _encoder_body,
        out_shape=jax.ShapeDtypeStruct((NPAD, D), jnp.float32),
    )(x, w, b, g, bb)


def _comb_one(relu, s_lo, s_hi, c, x, wl, bl, wr):
    r = 1.0 / jnp.maximum(c[:, 0:1], 1.0)
    agg = jnp.concatenate([s_lo, s_hi], axis=1) * r
    y = (jnp.dot(agg, wl, preferred_element_type=jnp.float32) + bl
         + jnp.dot(x, wr, preferred_element_type=jnp.float32))
    if relu:
        y = jnp.maximum(y, 0.0)
    return _mask_pad(y)


def _combine_body(relu, sl_ref, sr_ref, c_ref, x_ref, wl_ref, bl_ref,
                  wr_ref, o_ref):
    o_ref[...] = _comb_one(relu, sl_ref[...], sr_ref[...], c_ref[...],
                           x_ref[...], wl_ref[...], bl_ref[...], wr_ref[...])


def _combine(relu, s_lo, s_hi, c, x, wl, bl, wr):
    return pl.pallas_call(
        functools.partial(_combine_body, relu),
        out_shape=jax.ShapeDtypeStruct((NPAD, D), jnp.float32),
    )(s_lo, s_hi, c, x, wl, bl, wr)


_DOT_BLK = 2048


def _dot_body(u_ref, j_ref, o_ref):
    o_ref[...] = jnp.sum(u_ref[...] * j_ref[...], axis=1, keepdims=True)


def _dot(uf, jf):
    return pl.pallas_call(
        _dot_body,
        grid=(LP // _DOT_BLK,),
        in_specs=[pl.BlockSpec((_DOT_BLK, D), lambda i: (i, 0)),
                  pl.BlockSpec((_DOT_BLK, D), lambda i: (i, 0))],
        out_specs=pl.BlockSpec((_DOT_BLK, 1), lambda i: (i, 0)),
        out_shape=jax.ShapeDtypeStruct((LP, 1), jnp.float32),
    )(uf, jf)


def kernel(x_user, x_job, edge_index, rev_edge_index, edge_label_index,
           W_user, b_user, W_job, b_job, bn_g_user, bn_b_user, bn_g_job,
           bn_b_job, c1_rates_Wl, c1_rates_bl, c1_rates_Wr, c1_rev_Wl,
           c1_rev_bl, c1_rev_Wr, c2_rates_Wl, c2_rates_bl, c2_rates_Wr,
           c2_rev_Wl, c2_rev_bl, c2_rev_Wr):
    f32 = jnp.float32
    ei = edge_index.astype(jnp.int32)
    rev = rev_edge_index.astype(jnp.int32)
    eli = edge_label_index.astype(jnp.int32)

    xu = jnp.pad(x_user, ((0, NPAD - N), (0, 0)))
    xj = jnp.pad(x_job, ((0, NPAD - N), (0, 0)))
    u = _encoder(xu, W_user, b_user.reshape(1, D), bn_g_user.reshape(1, D),
                 bn_b_user.reshape(1, D))
    j = _encoder(xj, W_job, b_job.reshape(1, D), bn_g_job.reshape(1, D),
                 bn_b_job.reshape(1, D))

    pe = EP - E
    pad_src = jnp.arange(pe, dtype=jnp.int32) % N
    pad_dst = N + (jnp.arange(pe, dtype=jnp.int32) % (NPAD - N))
    srcA = jnp.concatenate([ei[0], pad_src])
    dstA = jnp.concatenate([ei[1], pad_dst])
    srcB = jnp.concatenate([rev[0] + NPAD, pad_src + NPAD])
    dstB = jnp.concatenate([rev[1], pad_dst])
    srci = jnp.stack([srcA, srcB]).reshape(NC, NS, NCHUNK_E, CHUNK)
    dsti = jnp.stack([dstA, dstB]).reshape(NC, NS, NCHUNK_E, CHUNK)

    zf = jnp.zeros((NPAD, HD), f32)
    zc = jnp.zeros((NPAD, LANES), f32)
    ones_h = jnp.ones((CHUNK, LANES), f32)

    tab1 = jnp.concatenate([u, j], axis=0)
    s1lo, s1hi, cnt = _segsum_wc(tab1[:, :HD], tab1[:, HD:], srci, dsti,
                                 zf, zc, ones_h)
    j1 = _combine(True, s1lo[0], s1hi[0], cnt[0], j, c1_rates_Wl,
                  c1_rates_bl.reshape(1, D), c1_rates_Wr)
    u1 = _combine(True, s1lo[1], s1hi[1], cnt[1], u, c1_rev_Wl,
                  c1_rev_bl.reshape(1, D), c1_rev_Wr)

    tab2 = jnp.concatenate([u1, j1], axis=0)
    s2lo, s2hi = _segsum_nc(tab2[:, :HD], tab2[:, HD:], srci, dsti, zf)
    j2 = _combine(False, s2lo[0], s2hi[0], cnt[0], j1, c2_rates_Wl,
                  c2_rates_bl.reshape(1, D), c2_rates_Wr)
    u2 = _combine(False, s2lo[1], s2hi[1], cnt[1], u1, c2_rev_Wl,
                  c2_rev_bl.reshape(1, D), c2_rev_Wr)
    tab3 = jnp.concatenate([u2, j2], axis=0)

    pla = LP - L
    pad_l = jnp.arange(pla, dtype=jnp.int32) % N
    l0 = jnp.concatenate([eli[0], pad_l])
    l1 = jnp.concatenate([eli[1] + NPAD, pad_l + NPAD])
    uf, jf = _labels(tab3, l0.reshape(NW, NCHUNK_L, CHUNK),
                     l1.reshape(NW, NCHUNK_L, CHUNK))
    dots = _dot(uf, jf)
    return dots[:L, 0]


# speedup vs baseline: 1.0234x; 1.0234x over previous
import functools

import jax
import jax.numpy as jnp
from jax import lax
from jax.experimental import pallas as pl
from jax.experimental.pallas import tpu as pltpu
from jax.experimental.pallas import tpu_sc as plsc

N = 10000
D = 128
E = 320000
L = 100000
NC, NS, LANES = 2, 16, 16
NW = NC * NS

ROWS_PER_TILE = 632
NPAD = NS * ROWS_PER_TILE
HD = 64
CHUNK = 128
NCHUNK_E = 160
EP = NS * NCHUNK_E * CHUNK
NCHUNK_L = 28
LP = NW * NCHUNK_L * CHUNK


def _make_segsum(with_counts):
    mesh = plsc.VectorSubcoreMesh(core_axis_name="c", subcore_axis_name="s")
    out_type = [jax.ShapeDtypeStruct((NC, NPAD, HD), jnp.float32)]
    NSLOT = 8
    DEPTH = 4
    G = 16
    NBLK = NCHUNK_E // G
    scratch = [pltpu.VMEM((G, CHUNK), jnp.int32) for _ in range(4)]
    scratch += [pltpu.VMEM((CHUNK, HD), jnp.float32) for _ in range(NSLOT)]
    scratch += [
        pltpu.VMEM_SHARED((NPAD, HD), jnp.float32),
        pltpu.SemaphoreType.DMA((NSLOT,)),
        pltpu.SemaphoreType.DMA((NSLOT,)),
        pltpu.SemaphoreType.DMA((2,)),
    ]
    if with_counts:
        out_type.append(jax.ShapeDtypeStruct((NC, NPAD, LANES), jnp.float32))
        scratch += [
            pltpu.VMEM((CHUNK, LANES), jnp.float32),
            pltpu.VMEM_SHARED((NPAD, LANES), jnp.float32),
            pltpu.SemaphoreType.DMA((NSLOT,)),
        ]

    def body(*args):
        if with_counts:
            (tab0, srci, dsti, zf, zc, ones_h, sums0, cnt,
             sv0, sv1, dv0, dv1,
             b0, b1, b2, b3, b4, b5, b6, b7, acc_sh, gsem, ssem, isem,
             ones_v, cnt_sh, csem) = args
        else:
            (tab0, srci, dsti, zf, sums0, sv0, sv1, dv0, dv1,
             b0, b1, b2, b3, b4, b5, b6, b7, acc_sh, gsem, ssem, isem) = args
        srcv = (sv0, sv1)
        dstv = (dv0, dv1)
        bufs = (b0, b1, b2, b3, b4, b5, b6, b7)
        cid = lax.axis_index("c")
        sid = lax.axis_index("s")
        base = pl.multiple_of(sid * ROWS_PER_TILE, 8)
        sl = pl.ds(base, ROWS_PER_TILE)

        def i_fire(blk, islot):
            off = pl.multiple_of(blk * G, 8)
            pltpu.async_copy(srci.at[cid, sid, pl.ds(off, G)], srcv[islot],
                             isem.at[islot])
            pltpu.async_copy(dsti.at[cid, sid, pl.ds(off, G)], dstv[islot],
                             isem.at[islot])

        def i_wait(islot):
            pltpu.make_async_copy(srci.at[0, 0, pl.ds(0, G)], srcv[islot],
                                  isem.at[islot]).wait()
            pltpu.make_async_copy(srci.at[0, 0, pl.ds(0, G)], dstv[islot],
                                  isem.at[islot]).wait()

        def run_half(tab, sums, do_cnt):
            def g_wait(b):
                pltpu.make_async_copy(tab.at[pl.ds(0, CHUNK)], bufs[b],
                                      gsem.at[b]).wait()

            def s_wait(b):
                pltpu.make_async_copy(tab.at[pl.ds(0, CHUNK)], bufs[b],
                                      ssem.at[b]).wait()

            def c_wait(b):
                pltpu.make_async_copy(zc.at[pl.ds(0, CHUNK)], ones_v,
                                      csem.at[b]).wait()

            def g_fire(islot, row, b):
                pltpu.async_copy(tab.at[srcv[islot].at[row]], bufs[b],
                                 gsem.at[b])

            def s_fire(islot, row, b):
                pltpu.async_copy(bufs[b], acc_sh.at[dstv[islot].at[row]],
                                 ssem.at[b], add=True)
                if do_cnt:
                    pltpu.async_copy(ones_v, cnt_sh.at[dstv[islot].at[row]],
                                     csem.at[b], add=True)

            pltpu.sync_copy(zf.at[sl], acc_sh.at[sl])
            if do_cnt:
                pltpu.sync_copy(zc.at[sl], cnt_sh.at[sl])
                pltpu.sync_copy(ones_h, ones_v)
            i_fire(0, 0)
            plsc.subcore_barrier()
            i_wait(0)
            for p in range(DEPTH):
                g_fire(0, p, p)

            def pair(bp, carry):
                for pb in range(2):
                    blk = bp * 2 + pb
                    for p in range(G):
                        b = p % NSLOT
                        nb = (b + DEPTH) % NSLOT
                        ci = blk * G + p
                        g_wait(b)
                        s_fire(pb, p, b)

                        @pl.when(ci >= DEPTH)
                        def _():
                            s_wait(nb)
                            if do_cnt:
                                c_wait(nb)

                        if p == 4:
                            @pl.when(blk <= NBLK - 2)
                            def _():
                                i_fire(blk + 1, 1 - pb)
                        if p == 11:
                            @pl.when(blk <= NBLK - 2)
                            def _():
                                i_wait(1 - pb)
                        tp = p + DEPTH
                        gslot, grow = (pb, tp) if tp < G else (1 - pb, tp - G)

                        @pl.when(ci <= NCHUNK_E - 1 - DEPTH)
                        def _():
                            g_fire(gslot, grow, nb)
                return carry

            lax.fori_loop(0, NBLK // 2, pair, 0)
            for b in range(NSLOT - DEPTH, NSLOT):
                s_wait(b)
                if do_cnt:
                    c_wait(b)
            plsc.subcore_barrier()
            pltpu.sync_copy(acc_sh.at[sl], sums.at[cid, sl])
            if do_cnt:
                pltpu.sync_copy(cnt_sh.at[sl], cnt.at[cid, sl])

        run_half(tab0, sums0, with_counts)

    return pl.kernel(body, out_type=tuple(out_type), mesh=mesh,
                     scratch_types=scratch,
                     compiler_params=pltpu.CompilerParams(
                         use_tc_tiling_on_sc=False))


_segsum_wc = _make_segsum(True)
_segsum_nc = _make_segsum(False)


def _make_labels():
    mesh = plsc.VectorSubcoreMesh(core_axis_name="c", subcore_axis_name="s")
    out_type = (jax.ShapeDtypeStruct((LP, D), jnp.float32),
                jax.ShapeDtypeStruct((LP, D), jnp.float32))
    NSLOT = 2
    DEPTH = 1
    scratch = [
        pltpu.VMEM((NCHUNK_L, CHUNK), jnp.int32),
        pltpu.VMEM((NCHUNK_L, CHUNK), jnp.int32),
    ]
    scratch += [pltpu.VMEM((CHUNK, D), jnp.float32) for _ in range(2 * NSLOT)]
    scratch += [
        pltpu.SemaphoreType.DMA((NSLOT,)),
        pltpu.SemaphoreType.DMA((NSLOT,)),
        pltpu.SemaphoreType.DMA((NSLOT,)),
        pltpu.SemaphoreType.DMA((NSLOT,)),
    ]

    def body(tab, l0i, l1i, uf, jf, l0_v, l1_v, ru0, ru1,
             rj0, rj1, gusem, gjsem, wusem, wjsem):
        cid = lax.axis_index("c")
        sid = lax.axis_index("s")
        w = sid * NC + cid
        rus = (ru0, ru1)
        rjs = (rj0, rj1)
        pltpu.sync_copy(l0i.at[w], l0_v)
        pltpu.sync_copy(l1i.at[w], l1_v)

        def g_fire(ci, b):
            pltpu.async_copy(tab.at[l0_v.at[ci]], rus[b], gusem.at[b])
            pltpu.async_copy(tab.at[l1_v.at[ci]], rjs[b], gjsem.at[b])

        def g_wait(b):
            pltpu.make_async_copy(tab.at[pl.ds(0, CHUNK)], rus[b],
                                  gusem.at[b]).wait()
            pltpu.make_async_copy(tab.at[pl.ds(0, CHUNK)], rjs[b],
                                  gjsem.at[b]).wait()

        def w_fire(ci, b):
            rb = pl.multiple_of(w * (NCHUNK_L * CHUNK) + ci * CHUNK, 8)
            pltpu.async_copy(rus[b], uf.at[pl.ds(rb, CHUNK)], wusem.at[b])
            pltpu.async_copy(rjs[b], jf.at[pl.ds(rb, CHUNK)], wjsem.at[b])

        def w_wait(b):
            pltpu.make_async_copy(tab.at[pl.ds(0, CHUNK)], rus[b],
                                  wusem.at[b]).wait()
            pltpu.make_async_copy(tab.at[pl.ds(0, CHUNK)], rjs[b],
                                  wjsem.at[b]).wait()

        for p in range(DEPTH):
            g_fire(p, p)

        def outer(io, carry):
            i = io * NSLOT
            for b in range(NSLOT):
                ci = i + b
                nb = (b + DEPTH) % NSLOT
                g_wait(b)
                w_fire(ci, b)

                @pl.when(ci >= DEPTH)
                def _():
                    w_wait(nb)

                @pl.when(ci <= NCHUNK_L - 1 - DEPTH)
                def _():
                    g_fire(ci + DEPTH, nb)
            return carry

        lax.fori_loop(0, NCHUNK_L // NSLOT, outer, 0)
        for b in range(NSLOT - DEPTH, NSLOT):
            w_wait(b)

    return pl.kernel(body, out_type=out_type, mesh=mesh,
                     scratch_types=scratch)


_labels = _make_labels()


def _mask_pad(y):
    rid = lax.broadcasted_iota(jnp.int32, y.shape, 0)
    return jnp.where(rid < N, y, 0.0)


def _enc_one(x, w, b, g, bb):
    h = jnp.dot(x, w, preferred_element_type=jnp.float32) + b
    hs = h[:N]
    mu = jnp.mean(hs, axis=0, keepdims=True)
    var = jnp.mean((hs - mu) ** 2, axis=0, keepdims=True)
    y = (h - mu) * lax.rsqrt(var + 1e-5) * g + bb
    return _mask_pad(jnp.maximum(y, 0.0))


def _encoder_body(x_ref, w_ref, b_ref, g_ref, bb_ref, o_ref):
    o_ref[...] = _enc_one(x_ref[...], w_ref[...], b_ref[...], g_ref[...],
                          bb_ref[...])


def _encoder(x, w, b, g, bb):
    return pl.pallas_call(
        _encoder_body,
        out_shape=jax.ShapeDtypeStruct((NPAD, D), jnp.float32),
    )(x, w, b, g, bb)


def _comb_one(relu, s_lo, s_hi, c, x, wl, bl, wr):
    r = 1.0 / jnp.maximum(c[:, 0:1], 1.0)
    agg = jnp.concatenate([s_lo, s_hi], axis=1) * r
    y = (jnp.dot(agg, wl, preferred_element_type=jnp.float32) + bl
         + jnp.dot(x, wr, preferred_element_type=jnp.float32))
    if relu:
        y = jnp.maximum(y, 0.0)
    return _mask_pad(y)


def _combine_body(relu, sl_ref, sr_ref, c_ref, x_ref, wl_ref, bl_ref,
                  wr_ref, o_ref):
    o_ref[...] = _comb_one(relu, sl_ref[...], sr_ref[...], c_ref[...],
                           x_ref[...], wl_ref[...], bl_ref[...], wr_ref[...])


def _combine(relu, s_lo, s_hi, c, x, wl, bl, wr):
    return pl.pallas_call(
        functools.partial(_combine_body, relu),
        out_shape=jax.ShapeDtypeStruct((NPAD, D), jnp.float32),
    )(s_lo, s_hi, c, x, wl, bl, wr)


_DOT_BLK = 2048


def _dot_body(u_ref, j_ref, o_ref):
    o_ref[...] = jnp.sum(u_ref[...] * j_ref[...], axis=1, keepdims=True)


def _dot(uf, jf):
    return pl.pallas_call(
        _dot_body,
        grid=(LP // _DOT_BLK,),
        in_specs=[pl.BlockSpec((_DOT_BLK, D), lambda i: (i, 0)),
                  pl.BlockSpec((_DOT_BLK, D), lambda i: (i, 0))],
        out_specs=pl.BlockSpec((_DOT_BLK, 1), lambda i: (i, 0)),
        out_shape=jax.ShapeDtypeStruct((LP, 1), jnp.float32),
    )(uf, jf)


def kernel(x_user, x_job, edge_index, rev_edge_index, edge_label_index,
           W_user, b_user, W_job, b_job, bn_g_user, bn_b_user, bn_g_job,
           bn_b_job, c1_rates_Wl, c1_rates_bl, c1_rates_Wr, c1_rev_Wl,
           c1_rev_bl, c1_rev_Wr, c2_rates_Wl, c2_rates_bl, c2_rates_Wr,
           c2_rev_Wl, c2_rev_bl, c2_rev_Wr):
    f32 = jnp.float32
    ei = edge_index.astype(jnp.int32)
    rev = rev_edge_index.astype(jnp.int32)
    eli = edge_label_index.astype(jnp.int32)

    xu = jnp.pad(x_user, ((0, NPAD - N), (0, 0)))
    xj = jnp.pad(x_job, ((0, NPAD - N), (0, 0)))
    u = _encoder(xu, W_user, b_user.reshape(1, D), bn_g_user.reshape(1, D),
                 bn_b_user.reshape(1, D))
    j = _encoder(xj, W_job, b_job.reshape(1, D), bn_g_job.reshape(1, D),
                 bn_b_job.reshape(1, D))

    pe = EP - E
    pad_src = jnp.arange(pe, dtype=jnp.int32) % N
    pad_dst = N + (jnp.arange(pe, dtype=jnp.int32) % (NPAD - N))
    srcA = jnp.concatenate([ei[0], pad_src])
    dstA = jnp.concatenate([ei[1], pad_dst])
    srcB = jnp.concatenate([rev[0] + NPAD, pad_src + NPAD])
    dstB = jnp.concatenate([rev[1], pad_dst])
    srci = jnp.stack([srcA, srcB]).reshape(NC, NS, NCHUNK_E, CHUNK)
    dsti = jnp.stack([dstA, dstB]).reshape(NC, NS, NCHUNK_E, CHUNK)

    zf = jnp.zeros((NPAD, HD), f32)
    zc = jnp.zeros((NPAD, LANES), f32)
    ones_h = jnp.ones((CHUNK, LANES), f32)

    tab1 = jnp.concatenate([u, j], axis=0)
    s1lo, cnt = _segsum_wc(tab1[:, :HD], srci, dsti, zf, zc, ones_h)
    (s1hi,) = _segsum_nc(tab1[:, HD:], srci, dsti, zf)
    j1 = _combine(True, s1lo[0], s1hi[0], cnt[0], j, c1_rates_Wl,
                  c1_rates_bl.reshape(1, D), c1_rates_Wr)
    u1 = _combine(True, s1lo[1], s1hi[1], cnt[1], u, c1_rev_Wl,
                  c1_rev_bl.reshape(1, D), c1_rev_Wr)

    tab2 = jnp.concatenate([u1, j1], axis=0)
    (s2lo,) = _segsum_nc(tab2[:, :HD], srci, dsti, zf)
    (s2hi,) = _segsum_nc(tab2[:, HD:], srci, dsti, zf)
    j2 = _combine(False, s2lo[0], s2hi[0], cnt[0], j1, c2_rates_Wl,
                  c2_rates_bl.reshape(1, D), c2_rates_Wr)
    u2 = _combine(False, s2lo[1], s2hi[1], cnt[1], u1, c2_rev_Wl,
                  c2_rev_bl.reshape(1, D), c2_rev_Wr)
    tab3 = jnp.concatenate([u2, j2], axis=0)

    pla = LP - L
    pad_l = jnp.arange(pla, dtype=jnp.int32) % N
    l0 = jnp.concatenate([eli[0], pad_l])
    l1 = jnp.concatenate([eli[1] + NPAD, pad_l + NPAD])
    uf, jf = _labels(tab3, l0.reshape(NW, NCHUNK_L, CHUNK),
                     l1.reshape(NW, NCHUNK_L, CHUNK))
    dots = _dot(uf, jf)
    return dots[:L, 0]


# speedup vs baseline: 1.0686x; 1.0442x over previous
import functools

import jax
import jax.numpy as jnp
from jax import lax
from jax.experimental import pallas as pl
from jax.experimental.pallas import tpu as pltpu
from jax.experimental.pallas import tpu_sc as plsc

N = 10000
D = 128
E = 320000
L = 100000
NC, NS, LANES = 2, 16, 16
NW = NC * NS

ROWS_PER_TILE = 632
NPAD = NS * ROWS_PER_TILE
HD = 64
CHUNK = 128
NCHUNK_E = 160
EP = NS * NCHUNK_E * CHUNK
NCHUNK_L = 28
LP = NW * NCHUNK_L * CHUNK


def _make_segsum(with_counts):
    mesh = plsc.VectorSubcoreMesh(core_axis_name="c", subcore_axis_name="s")
    out_type = [jax.ShapeDtypeStruct((NC, NPAD, HD), jnp.float32)]
    NSLOT = 8
    DEPTH = 4
    G = 16
    NBLK = NCHUNK_E // G
    scratch = [pltpu.VMEM((G, CHUNK), jnp.int32) for _ in range(4)]
    scratch += [pltpu.VMEM((CHUNK, HD), jnp.float32) for _ in range(NSLOT)]
    scratch += [
        pltpu.VMEM_SHARED((NPAD, HD), jnp.float32),
        pltpu.SemaphoreType.DMA((NSLOT,)),
        pltpu.SemaphoreType.DMA((NSLOT,)),
        pltpu.SemaphoreType.DMA((2,)),
    ]
    if with_counts:
        out_type.append(jax.ShapeDtypeStruct((NC, NPAD, LANES), jnp.float32))
        scratch += [
            pltpu.VMEM((CHUNK, LANES), jnp.float32),
            pltpu.VMEM_SHARED((NPAD, LANES), jnp.float32),
            pltpu.SemaphoreType.DMA((NSLOT,)),
        ]

    def body(*args):
        if with_counts:
            (tab0, srci, dsti, zf, zc, ones_h, sums0, cnt,
             sv0, sv1, dv0, dv1,
             b0, b1, b2, b3, b4, b5, b6, b7, acc_sh, gsem, ssem, isem,
             ones_v, cnt_sh, csem) = args
        else:
            (tab0, srci, dsti, zf, sums0, sv0, sv1, dv0, dv1,
             b0, b1, b2, b3, b4, b5, b6, b7, acc_sh, gsem, ssem, isem) = args
        srcv = (sv0, sv1)
        dstv = (dv0, dv1)
        bufs = (b0, b1, b2, b3, b4, b5, b6, b7)
        cid = lax.axis_index("c")
        sid = lax.axis_index("s")
        base = pl.multiple_of(sid * ROWS_PER_TILE, 8)
        sl = pl.ds(base, ROWS_PER_TILE)

        def i_fire(blk, islot):
            off = pl.multiple_of(blk * G, 8)
            pltpu.async_copy(srci.at[cid, sid, pl.ds(off, G)], srcv[islot],
                             isem.at[islot])
            pltpu.async_copy(dsti.at[cid, sid, pl.ds(off, G)], dstv[islot],
                             isem.at[islot])

        def i_wait(islot):
            pltpu.make_async_copy(srci.at[0, 0, pl.ds(0, G)], srcv[islot],
                                  isem.at[islot]).wait()
            pltpu.make_async_copy(srci.at[0, 0, pl.ds(0, G)], dstv[islot],
                                  isem.at[islot]).wait()

        def run_half(tab, sums, do_cnt):
            def g_wait(b):
                pltpu.make_async_copy(tab.at[pl.ds(0, CHUNK)], bufs[b],
                                      gsem.at[b]).wait()

            def s_wait(b):
                pltpu.make_async_copy(tab.at[pl.ds(0, CHUNK)], bufs[b],
                                      ssem.at[b]).wait()

            def c_wait(b):
                pltpu.make_async_copy(zc.at[pl.ds(0, CHUNK)], ones_v,
                                      csem.at[b]).wait()

            def g_fire(islot, row, b):
                pltpu.async_copy(tab.at[srcv[islot].at[row]], bufs[b],
                                 gsem.at[b])

            def s_fire(islot, row, b):
                pltpu.async_copy(bufs[b], acc_sh.at[dstv[islot].at[row]],
                                 ssem.at[b], add=True)
                if do_cnt:
                    pltpu.async_copy(ones_v, cnt_sh.at[dstv[islot].at[row]],
                                     csem.at[b], add=True)

            pltpu.sync_copy(zf.at[sl], acc_sh.at[sl])
            if do_cnt:
                pltpu.sync_copy(zc.at[sl], cnt_sh.at[sl])
                pltpu.sync_copy(ones_h, ones_v)
            i_fire(0, 0)
            plsc.subcore_barrier()
            i_wait(0)
            for p in range(DEPTH):
                g_fire(0, p, p)

            def pair(bp, carry):
                for pb in range(2):
                    blk = bp * 2 + pb
                    for p in range(G):
                        b = p % NSLOT
                        nb = (b + DEPTH) % NSLOT
                        ci = blk * G + p
                        g_wait(b)
                        s_fire(pb, p, b)

                        @pl.when(ci >= DEPTH)
                        def _():
                            s_wait(nb)
                            if do_cnt:
                                c_wait(nb)

                        if p == 4:
                            @pl.when(blk <= NBLK - 2)
                            def _():
                                i_fire(blk + 1, 1 - pb)
                        if p == 11:
                            @pl.when(blk <= NBLK - 2)
                            def _():
                                i_wait(1 - pb)
                        tp = p + DEPTH
                        gslot, grow = (pb, tp) if tp < G else (1 - pb, tp - G)

                        @pl.when(ci <= NCHUNK_E - 1 - DEPTH)
                        def _():
                            g_fire(gslot, grow, nb)
                return carry

            lax.fori_loop(0, NBLK // 2, pair, 0)
            for b in range(NSLOT - DEPTH, NSLOT):
                s_wait(b)
                if do_cnt:
                    c_wait(b)
            plsc.subcore_barrier()
            pltpu.sync_copy(acc_sh.at[sl], sums.at[cid, sl])
            if do_cnt:
                pltpu.sync_copy(cnt_sh.at[sl], cnt.at[cid, sl])

        run_half(tab0, sums0, with_counts)

    return pl.kernel(body, out_type=tuple(out_type), mesh=mesh,
                     scratch_types=scratch,
                     compiler_params=pltpu.CompilerParams(
                         use_tc_tiling_on_sc=False))


_segsum_wc = _make_segsum(True)
_segsum_nc = _make_segsum(False)


def _make_labels():
    mesh = plsc.VectorSubcoreMesh(core_axis_name="c", subcore_axis_name="s")
    out_type = (jax.ShapeDtypeStruct((LP, D), jnp.float32),
                jax.ShapeDtypeStruct((LP, D), jnp.float32))
    NSLOT = 2
    DEPTH = 1
    scratch = [
        pltpu.VMEM((NCHUNK_L, CHUNK), jnp.int32),
        pltpu.VMEM((NCHUNK_L, CHUNK), jnp.int32),
    ]
    scratch += [pltpu.VMEM((CHUNK, D), jnp.float32) for _ in range(2 * NSLOT)]
    scratch += [
        pltpu.SemaphoreType.DMA((NSLOT,)),
        pltpu.SemaphoreType.DMA((NSLOT,)),
        pltpu.SemaphoreType.DMA((NSLOT,)),
        pltpu.SemaphoreType.DMA((NSLOT,)),
    ]

    def body(tab, l0i, l1i, uf, jf, l0_v, l1_v, ru0, ru1,
             rj0, rj1, gusem, gjsem, wusem, wjsem):
        cid = lax.axis_index("c")
        sid = lax.axis_index("s")
        w = sid * NC + cid
        rus = (ru0, ru1)
        rjs = (rj0, rj1)
        pltpu.sync_copy(l0i.at[w], l0_v)
        pltpu.sync_copy(l1i.at[w], l1_v)

        def g_fire(ci, b):
            pltpu.async_copy(tab.at[l0_v.at[ci]], rus[b], gusem.at[b])
            pltpu.async_copy(tab.at[l1_v.at[ci]], rjs[b], gjsem.at[b])

        def g_wait(b):
            pltpu.make_async_copy(tab.at[pl.ds(0, CHUNK)], rus[b],
                                  gusem.at[b]).wait()
            pltpu.make_async_copy(tab.at[pl.ds(0, CHUNK)], rjs[b],
                                  gjsem.at[b]).wait()

        def w_fire(ci, b):
            rb = pl.multiple_of(w * (NCHUNK_L * CHUNK) + ci * CHUNK, 8)
            pltpu.async_copy(rus[b], uf.at[pl.ds(rb, CHUNK)], wusem.at[b])
            pltpu.async_copy(rjs[b], jf.at[pl.ds(rb, CHUNK)], wjsem.at[b])

        def w_wait(b):
            pltpu.make_async_copy(tab.at[pl.ds(0, CHUNK)], rus[b],
                                  wusem.at[b]).wait()
            pltpu.make_async_copy(tab.at[pl.ds(0, CHUNK)], rjs[b],
                                  wjsem.at[b]).wait()

        for p in range(DEPTH):
            g_fire(p, p)

        def outer(io, carry):
            i = io * NSLOT
            for b in range(NSLOT):
                ci = i + b
                nb = (b + DEPTH) % NSLOT
                g_wait(b)
                w_fire(ci, b)

                @pl.when(ci >= DEPTH)
                def _():
                    w_wait(nb)

                @pl.when(ci <= NCHUNK_L - 1 - DEPTH)
                def _():
                    g_fire(ci + DEPTH, nb)
            return carry

        lax.fori_loop(0, NCHUNK_L // NSLOT, outer, 0)
        for b in range(NSLOT - DEPTH, NSLOT):
            w_wait(b)

    return pl.kernel(body, out_type=out_type, mesh=mesh,
                     scratch_types=scratch)


_labels = _make_labels()


def _mask_pad(y):
    rid = lax.broadcasted_iota(jnp.int32, y.shape, 0)
    return jnp.where(rid < N, y, 0.0)


def _enc_one(x, w, b, g, bb):
    h = jnp.dot(x, w, preferred_element_type=jnp.float32) + b
    hs = h[:N]
    mu = jnp.mean(hs, axis=0, keepdims=True)
    var = jnp.mean((hs - mu) ** 2, axis=0, keepdims=True)
    y = (h - mu) * lax.rsqrt(var + 1e-5) * g + bb
    return _mask_pad(jnp.maximum(y, 0.0))


def _encoder_body(x_ref, w_ref, b_ref, g_ref, bb_ref, o_ref):
    o_ref[...] = _enc_one(x_ref[...], w_ref[...], b_ref[...], g_ref[...],
                          bb_ref[...])


def _encoder(x, w, b, g, bb):
    return pl.pallas_call(
        _encoder_body,
        out_shape=jax.ShapeDtypeStruct((NPAD, D), jnp.float32),
    )(x, w, b, g, bb)


def _comb_one(relu, s_lo, s_hi, c, x, wl, bl, wr):
    r = 1.0 / jnp.maximum(c[:, 0:1], 1.0)
    agg = jnp.concatenate([s_lo, s_hi], axis=1) * r
    y = (jnp.dot(agg, wl, preferred_element_type=jnp.float32) + bl
         + jnp.dot(x, wr, preferred_element_type=jnp.float32))
    if relu:
        y = jnp.maximum(y, 0.0)
    return _mask_pad(y)


def _combine_body(relu, sl_ref, sr_ref, c_ref, tab_ref, wla_ref, bla_ref,
                  wra_ref, wlb_ref, blb_ref, wrb_ref, o_ref):
    sl = sl_ref[...]
    sr = sr_ref[...]
    c = c_ref[...]
    tab = tab_ref[...]
    jn = _comb_one(relu, sl[0], sr[0], c[0], tab[NPAD:], wla_ref[...],
                   bla_ref[...], wra_ref[...])
    un = _comb_one(relu, sl[1], sr[1], c[1], tab[:NPAD], wlb_ref[...],
                   blb_ref[...], wrb_ref[...])
    o_ref[...] = jnp.concatenate([un, jn], axis=0)


def _combine(relu, s_lo, s_hi, c, tab, wla, bla, wra, wlb, blb, wrb):
    return pl.pallas_call(
        functools.partial(_combine_body, relu),
        out_shape=jax.ShapeDtypeStruct((2 * NPAD, D), jnp.float32),
    )(s_lo, s_hi, c, tab, wla, bla, wra, wlb, blb, wrb)


_DOT_BLK = 2048


def _dot_body(u_ref, j_ref, o_ref):
    o_ref[...] = jnp.sum(u_ref[...] * j_ref[...], axis=1, keepdims=True)


def _dot(uf, jf):
    return pl.pallas_call(
        _dot_body,
        grid=(LP // _DOT_BLK,),
        in_specs=[pl.BlockSpec((_DOT_BLK, D), lambda i: (i, 0)),
                  pl.BlockSpec((_DOT_BLK, D), lambda i: (i, 0))],
        out_specs=pl.BlockSpec((_DOT_BLK, 1), lambda i: (i, 0)),
        out_shape=jax.ShapeDtypeStruct((LP, 1), jnp.float32),
    )(uf, jf)


def kernel(x_user, x_job, edge_index, rev_edge_index, edge_label_index,
           W_user, b_user, W_job, b_job, bn_g_user, bn_b_user, bn_g_job,
           bn_b_job, c1_rates_Wl, c1_rates_bl, c1_rates_Wr, c1_rev_Wl,
           c1_rev_bl, c1_rev_Wr, c2_rates_Wl, c2_rates_bl, c2_rates_Wr,
           c2_rev_Wl, c2_rev_bl, c2_rev_Wr):
    f32 = jnp.float32
    ei = edge_index.astype(jnp.int32)
    rev = rev_edge_index.astype(jnp.int32)
    eli = edge_label_index.astype(jnp.int32)

    xu = jnp.pad(x_user, ((0, NPAD - N), (0, 0)))
    xj = jnp.pad(x_job, ((0, NPAD - N), (0, 0)))
    u = _encoder(xu, W_user, b_user.reshape(1, D), bn_g_user.reshape(1, D),
                 bn_b_user.reshape(1, D))
    j = _encoder(xj, W_job, b_job.reshape(1, D), bn_g_job.reshape(1, D),
                 bn_b_job.reshape(1, D))

    pe = EP - E
    pad_src = jnp.arange(pe, dtype=jnp.int32) % N
    pad_dst = N + (jnp.arange(pe, dtype=jnp.int32) % (NPAD - N))
    srcA = jnp.concatenate([ei[0], pad_src])
    dstA = jnp.concatenate([ei[1], pad_dst])
    srcB = jnp.concatenate([rev[0] + NPAD, pad_src + NPAD])
    dstB = jnp.concatenate([rev[1], pad_dst])
    srci = jnp.stack([srcA, srcB]).reshape(NC, NS, NCHUNK_E, CHUNK)
    dsti = jnp.stack([dstA, dstB]).reshape(NC, NS, NCHUNK_E, CHUNK)

    zf = jnp.zeros((NPAD, HD), f32)
    zc = jnp.zeros((NPAD, LANES), f32)
    ones_h = jnp.ones((CHUNK, LANES), f32)

    tab1 = jnp.concatenate([u, j], axis=0)
    s1lo, cnt = _segsum_wc(tab1[:, :HD], srci, dsti, zf, zc, ones_h)
    (s1hi,) = _segsum_nc(tab1[:, HD:], srci, dsti, zf)
    tab2 = _combine(True, s1lo, s1hi, cnt, tab1,
                    c1_rates_Wl, c1_rates_bl.reshape(1, D), c1_rates_Wr,
                    c1_rev_Wl, c1_rev_bl.reshape(1, D), c1_rev_Wr)

    (s2lo,) = _segsum_nc(tab2[:, :HD], srci, dsti, zf)
    (s2hi,) = _segsum_nc(tab2[:, HD:], srci, dsti, zf)
    tab3 = _combine(False, s2lo, s2hi, cnt, tab2,
                    c2_rates_Wl, c2_rates_bl.reshape(1, D), c2_rates_Wr,
                    c2_rev_Wl, c2_rev_bl.reshape(1, D), c2_rev_Wr)

    pla = LP - L
    pad_l = jnp.arange(pla, dtype=jnp.int32) % N
    l0 = jnp.concatenate([eli[0], pad_l])
    l1 = jnp.concatenate([eli[1] + NPAD, pad_l + NPAD])
    uf, jf = _labels(tab3, l0.reshape(NW, NCHUNK_L, CHUNK),
                     l1.reshape(NW, NCHUNK_L, CHUNK))
    dots = _dot(uf, jf)
    return dots[:L, 0]
